# baseline (speedup 1.0000x reference)
import functools

import jax
import jax.numpy as jnp
from jax import lax
from jax.experimental import pallas as pl
from jax.experimental.pallas import tpu as pltpu
from jax.experimental.pallas import tpu_sc as plsc

_N = 65536
_D = 64
_P = 65536
_R = 16384
_L1_COEFF = 0.001 / float(_D * _D)

_NC = 2
_NS = 16
_NW = _NC * _NS
_CH = 128
_LANES = 16

_SC_PARAMS = pltpu.CompilerParams(
    needs_layout_passes=False, use_tc_tiling_on_sc=False)
_SC_MESH = dict(core_axis_name="c", subcore_axis_name="s")


def _sqrt16(x):
    i = lax.bitcast_convert_type(x, jnp.int32)
    i = jnp.int32(0x1FBD1DF5) + lax.shift_right_logical(i, 1)
    y = lax.bitcast_convert_type(i, jnp.float32)
    for _ in range(3):
        y = 0.5 * (y + x / y)
    return y


def _cols():
    lane = lax.iota(jnp.int32, _LANES)
    return [(lane + j) & (_D - 1) for j in range(_D)]


def _pipelined(n, fire, drain, compute, carry, nbuf):
    for b in range(nbuf):
        fire(b, b)

    def body(k, cr):
        c = k * nbuf
        for b in range(nbuf):
            drain(b)
            cr = compute(b, c + b, cr)
            fire(c + nbuf + b, b)
        return cr

    carry = lax.fori_loop(0, n // nbuf - 1, body, carry)
    for b in range(nbuf):
        drain(b)
        carry = compute(b, n - nbuf + b, carry)
    return carry


@functools.partial(
    pl.kernel,
    mesh=plsc.VectorSubcoreMesh(**_SC_MESH),
    compiler_params=_SC_PARAMS,
    out_type=jax.ShapeDtypeStruct((_NW, 2 * _LANES), jnp.float32),
    scratch_types=[
        pltpu.VMEM((_P // _NW,), jnp.int32),
        pltpu.VMEM((_P // _NW,), jnp.int32),
        pltpu.VMEM((_CH, _D), jnp.float32),
        pltpu.VMEM((_CH, _D), jnp.float32),
        pltpu.VMEM((_CH, _D), jnp.float32),
        pltpu.VMEM((_CH, _D), jnp.float32),
        pltpu.VMEM((_CH, _D), jnp.float32),
        pltpu.VMEM((_CH, _D), jnp.float32),
        pltpu.VMEM((_CH, _D), jnp.float32),
        pltpu.VMEM((_CH, _D), jnp.float32),
        pltpu.VMEM((2 * _LANES,), jnp.float32),
        pltpu.SemaphoreType.DMA,
        pltpu.SemaphoreType.DMA,
        pltpu.SemaphoreType.DMA,
        pltpu.SemaphoreType.DMA,
    ],
)
def _sc_states_losses(states_hbm, disa_hbm, disb_hbm, refa_hbm, refb_hbm,
                      out_hbm, ia_all, ib_all, bufa0, bufa1, bufa2, bufa3,
                      bufb0, bufb1, bufb2, bufb3,
                      accs, sem0, sem1, sem2, sem3):
    wid = lax.axis_index("s") * _NC + lax.axis_index("c")
    lane = lax.iota(jnp.int32, _LANES)
    zero = jnp.zeros((_LANES,), jnp.float32)
    bufa = (bufa0, bufa1, bufa2, bufa3)
    bufb = (bufb0, bufb1, bufb2, bufb3)
    sems = (sem0, sem1, sem2, sem3)

    def load_idx(a_hbm, b_hbm, per_w):
        pltpu.sync_copy(a_hbm.at[pl.ds(wid * per_w, per_w)],
                        ia_all.at[pl.ds(0, per_w)])
        pltpu.sync_copy(b_hbm.at[pl.ds(wid * per_w, per_w)],
                        ib_all.at[pl.ds(0, per_w)])

    def fire(c, s):
        ia = ia_all.at[pl.ds(c * _CH, _CH)]
        ib = ib_all.at[pl.ds(c * _CH, _CH)]
        pltpu.async_copy(states_hbm.at[ia], bufa[s], sems[s])
        pltpu.async_copy(states_hbm.at[ib], bufb[s], sems[s])

    def drain(s):
        ia = ia_all.at[pl.ds(0, _CH)]
        pltpu.make_async_copy(states_hbm.at[ia], bufa[s], sems[s]).wait()
        pltpu.make_async_copy(states_hbm.at[ia], bufb[s], sems[s]).wait()

    cols = _cols()

    def dist2(rows, b1, b2):
        acc = [None] * 4
        for j in range(_D):
            dv = (plsc.load_gather(b1, [rows, cols[j]])
                  - plsc.load_gather(b2, [rows, cols[j]]))
            q = j % 4
            acc[q] = dv * dv if acc[q] is None else acc[q] + dv * dv
        return (acc[0] + acc[1]) + (acc[2] + acc[3])

    def dis_compute(s, c, acc):
        def grp(g, a):
            return a + jnp.exp(-dist2(g * _LANES + lane, bufa[s], bufb[s]))

        return lax.fori_loop(0, _CH // _LANES, grp, acc)

    def ref_compute(s, c, acc):
        def grp(g, a):
            return a + dist2(g * _LANES + lane, bufa[s], bufb[s])

        return lax.fori_loop(0, _CH // _LANES, grp, acc)

    load_idx(disa_hbm, disb_hbm, _P // _NW)
    acc_caus = _pipelined((_P // _NW) // _CH, fire, drain, dis_compute, zero,
                          nbuf=4)
    load_idx(refa_hbm, refb_hbm, _R // _NW)
    acc_fix = _pipelined((_R // _NW) // _CH, fire, drain, ref_compute, zero,
                         nbuf=4)

    accs[pl.ds(0, _LANES)] = acc_caus
    accs[pl.ds(_LANES, _LANES)] = acc_fix
    pltpu.sync_copy(accs, out_hbm.at[wid])


_CH2 = 64


@functools.partial(
    pl.kernel,
    mesh=plsc.VectorSubcoreMesh(**_SC_MESH),
    compiler_params=_SC_PARAMS,
    out_type=jax.ShapeDtypeStruct((_NW, 2 * _LANES), jnp.float32),
    scratch_types=(
        [pltpu.VMEM((_P // _NW,), jnp.int32)] * 2
        + [pltpu.VMEM((_CH2, _D), jnp.float32)] * 16
        + [pltpu.VMEM((2 * _LANES,), jnp.float32)]
        + [pltpu.SemaphoreType.DMA] * 4
    ),
)
def _sc_pairdiff_losses(states_hbm, nstates_hbm, saa_hbm, sab_hbm,
                        out_hbm, ia_all, ib_all, *rest):
    bufs = rest[:16]
    accs = rest[16]
    sems = rest[17:21]
    wid = lax.axis_index("s") * _NC + lax.axis_index("c")
    lane = lax.iota(jnp.int32, _LANES)
    zero = jnp.zeros((_LANES,), jnp.float32)
    bufa = bufs[0:4]
    bufb = bufs[4:8]
    bufc = bufs[8:12]
    bufd = bufs[12:16]
    per_w = _P // _NW

    pltpu.sync_copy(saa_hbm.at[pl.ds(wid * per_w, per_w)], ia_all)
    pltpu.sync_copy(sab_hbm.at[pl.ds(wid * per_w, per_w)], ib_all)

    def fire(c, s):
        ia = ia_all.at[pl.ds(c * _CH2, _CH2)]
        ib = ib_all.at[pl.ds(c * _CH2, _CH2)]
        pltpu.async_copy(states_hbm.at[ia], bufa[s], sems[s])
        pltpu.async_copy(states_hbm.at[ib], bufb[s], sems[s])
        pltpu.async_copy(nstates_hbm.at[ia], bufc[s], sems[s])
        pltpu.async_copy(nstates_hbm.at[ib], bufd[s], sems[s])

    def drain(s):
        ia = ia_all.at[pl.ds(0, _CH2)]
        pltpu.make_async_copy(states_hbm.at[ia], bufa[s], sems[s]).wait()
        pltpu.make_async_copy(states_hbm.at[ia], bufb[s], sems[s]).wait()
        pltpu.make_async_copy(states_hbm.at[ia], bufc[s], sems[s]).wait()
        pltpu.make_async_copy(states_hbm.at[ia], bufd[s], sems[s]).wait()

    cols = _cols()

    def sa_compute(s, c, carry):
        ba, bb, bc, bd = bufa[s], bufb[s], bufc[s], bufd[s]

        def grp(g, cr):
            ap, ar = cr
            rows = g * _LANES + lane
            f1 = [None] * 2
            f2 = [None] * 2
            f3 = [None] * 2
            f4 = [None] * 2
            for j in range(_D):
                sa_ = plsc.load_gather(ba, [rows, cols[j]])
                sb_ = plsc.load_gather(bb, [rows, cols[j]])
                na_ = plsc.load_gather(bc, [rows, cols[j]])
                nb_ = plsc.load_gather(bd, [rows, cols[j]])
                ds = sa_ - sb_
                da = na_ - sa_
                db = nb_ - sb_
                dd = da - db
                q = j % 2
                if f1[q] is None:
                    f1[q], f2[q], f3[q], f4[q] = ds * ds, dd * dd, da * da, db * db
                else:
                    f1[q] = f1[q] + ds * ds
                    f2[q] = f2[q] + dd * dd
                    f3[q] = f3[q] + da * da
                    f4[q] = f4[q] + db * db
            n2s = f1[0] + f1[1]
            n2d = f2[0] + f2[1]
            n2a = f3[0] + f3[1]
            n2b = f4[0] + f4[1]
            dsn = _sqrt16(n2a) - _sqrt16(n2b)
            ap = ap + dsn * dsn
            ar = ar + jnp.exp(-n2s) * n2d
            return (ap, ar)

        return lax.fori_loop(0, _CH2 // _LANES, grp, carry)

    acc_prop, acc_rep = _pipelined(per_w // _CH2, fire, drain, sa_compute,
                                   (zero, zero), nbuf=4)

    accs[pl.ds(0, _LANES)] = acc_prop
    accs[pl.ds(_LANES, _LANES)] = acc_rep
    pltpu.sync_copy(accs, out_hbm.at[wid])


_TBLK = 131072


def _tc_body(s_ref, ns_ref, w_ref, part_ref):
    d = ns_ref[...] - s_ref[...]
    tot = jnp.sum(d * d)
    wsum = jnp.sum(jnp.abs(w_ref[...]))
    lanes = lax.broadcasted_iota(jnp.int32, (1, 8, 128), 2)
    part_ref[...] = jnp.where(lanes == 0, tot, jnp.where(lanes == 1, wsum, 0.0))


_tc_dense = pl.pallas_call(
    _tc_body,
    grid=(_N * _D // _TBLK,),
    in_specs=[
        pl.BlockSpec((_TBLK,), lambda i: (i,)),
        pl.BlockSpec((_TBLK,), lambda i: (i,)),
        pl.BlockSpec((_D, _D), lambda i: (0, 0)),
    ],
    out_specs=pl.BlockSpec((1, 8, 128), lambda i: (i, 0, 0)),
    out_shape=jax.ShapeDtypeStruct((_N * _D // _TBLK, 8, 128), jnp.float32),
)


def kernel(states, next_states, dissimilar_pairs, same_actions_pairs,
           ref_point_pairs, similar_pairs, W):
    del similar_pairs
    sc1 = _sc_states_losses(
        states,
        dissimilar_pairs[:, 0], dissimilar_pairs[:, 1],
        ref_point_pairs[:, 0], ref_point_pairs[:, 1],
    )
    sc2 = _sc_pairdiff_losses(
        states, next_states,
        same_actions_pairs[:, 0], same_actions_pairs[:, 1],
    )
    part = _tc_dense(states.reshape(-1), next_states.reshape(-1), W)
    s1 = jnp.sum(sc1.reshape(_NW, 2, _LANES), axis=(0, 2))
    s2 = jnp.sum(sc2.reshape(_NW, 2, _LANES), axis=(0, 2))
    temp_coherence = jnp.sum(part[:, 0, 0]) / _N
    l1 = part[0, 0, 1]
    return (temp_coherence
            + s1[0] / _P
            + s2[0] / _P
            + s2[1] / _P
            + s1[1] / _R
            + _L1_COEFF * l1)


# speedup vs baseline: 1.0632x; 1.0632x over previous
import functools

import jax
import jax.numpy as jnp
from jax import lax
from jax.experimental import pallas as pl
from jax.experimental.pallas import tpu as pltpu
from jax.experimental.pallas import tpu_sc as plsc

_N = 65536
_D = 64
_P = 65536
_R = 16384
_L1_COEFF = 0.001 / float(_D * _D)

_NC = 2
_NS = 16
_NW = _NC * _NS
_CH = 128
_LANES = 16

_SC_PARAMS = pltpu.CompilerParams(
    needs_layout_passes=False, use_tc_tiling_on_sc=False)
_SC_MESH = dict(core_axis_name="c", subcore_axis_name="s")


def _sqrt16(x):
    i = lax.bitcast_convert_type(x, jnp.int32)
    i = jnp.int32(0x1FBD1DF5) + lax.shift_right_logical(i, 1)
    y = lax.bitcast_convert_type(i, jnp.float32)
    for _ in range(3):
        y = 0.5 * (y + x / y)
    return y


def _cols():
    lane = lax.iota(jnp.int32, _LANES)
    return [(lane + j) & (_D - 1) for j in range(_D)]


def _pipelined(n, fire, drain, compute, carry, nbuf):
    for b in range(nbuf):
        fire(b, b)

    def body(k, cr):
        c = k * nbuf
        for b in range(nbuf):
            drain(b)
            cr = compute(b, c + b, cr)
            fire(c + nbuf + b, b)
        return cr

    carry = lax.fori_loop(0, n // nbuf - 1, body, carry)
    for b in range(nbuf):
        drain(b)
        carry = compute(b, n - nbuf + b, carry)
    return carry


@functools.partial(
    pl.kernel,
    mesh=plsc.VectorSubcoreMesh(**_SC_MESH),
    compiler_params=_SC_PARAMS,
    out_type=jax.ShapeDtypeStruct((_NW, 2 * _LANES), jnp.float32),
    scratch_types=[
        pltpu.VMEM((_P // _NW,), jnp.int32),
        pltpu.VMEM((_P // _NW,), jnp.int32),
        pltpu.VMEM((_CH, _D), jnp.float32),
        pltpu.VMEM((_CH, _D), jnp.float32),
        pltpu.VMEM((_CH, _D), jnp.float32),
        pltpu.VMEM((_CH, _D), jnp.float32),
        pltpu.VMEM((_CH, _D), jnp.float32),
        pltpu.VMEM((_CH, _D), jnp.float32),
        pltpu.VMEM((_CH, _D), jnp.float32),
        pltpu.VMEM((_CH, _D), jnp.float32),
        pltpu.VMEM((2 * _LANES,), jnp.float32),
        pltpu.SemaphoreType.DMA,
        pltpu.SemaphoreType.DMA,
        pltpu.SemaphoreType.DMA,
        pltpu.SemaphoreType.DMA,
    ],
)
def _sc_states_losses(states_hbm, disa_hbm, disb_hbm, refa_hbm, refb_hbm,
                      out_hbm, ia_all, ib_all, bufa0, bufa1, bufa2, bufa3,
                      bufb0, bufb1, bufb2, bufb3,
                      accs, sem0, sem1, sem2, sem3):
    wid = lax.axis_index("s") * _NC + lax.axis_index("c")
    lane = lax.iota(jnp.int32, _LANES)
    zero = jnp.zeros((_LANES,), jnp.float32)
    bufa = (bufa0, bufa1, bufa2, bufa3)
    bufb = (bufb0, bufb1, bufb2, bufb3)
    sems = (sem0, sem1, sem2, sem3)

    def load_idx(a_hbm, b_hbm, per_w):
        pltpu.sync_copy(a_hbm.at[pl.ds(wid * per_w, per_w)],
                        ia_all.at[pl.ds(0, per_w)])
        pltpu.sync_copy(b_hbm.at[pl.ds(wid * per_w, per_w)],
                        ib_all.at[pl.ds(0, per_w)])

    def fire(c, s):
        ia = ia_all.at[pl.ds(c * _CH, _CH)]
        ib = ib_all.at[pl.ds(c * _CH, _CH)]
        pltpu.async_copy(states_hbm.at[ia], bufa[s], sems[s])
        pltpu.async_copy(states_hbm.at[ib], bufb[s], sems[s])

    def drain(s):
        ia = ia_all.at[pl.ds(0, _CH)]
        pltpu.make_async_copy(states_hbm.at[ia], bufa[s], sems[s]).wait()
        pltpu.make_async_copy(states_hbm.at[ia], bufb[s], sems[s]).wait()

    cols = _cols()

    def dist2(rows, b1, b2):
        acc = [None] * 4
        for j in range(_D):
            dv = (plsc.load_gather(b1, [rows, cols[j]])
                  - plsc.load_gather(b2, [rows, cols[j]]))
            q = j % 4
            acc[q] = dv * dv if acc[q] is None else acc[q] + dv * dv
        return (acc[0] + acc[1]) + (acc[2] + acc[3])

    def dis_compute(s, c, acc):
        def grp(g, a):
            return a + jnp.exp(-dist2(g * _LANES + lane, bufa[s], bufb[s]))

        return lax.fori_loop(0, _CH // _LANES, grp, acc)

    def ref_compute(s, c, acc):
        def grp(g, a):
            return a + dist2(g * _LANES + lane, bufa[s], bufb[s])

        return lax.fori_loop(0, _CH // _LANES, grp, acc)

    load_idx(disa_hbm, disb_hbm, _P // _NW)
    acc_caus = _pipelined((_P // _NW) // _CH, fire, drain, dis_compute, zero,
                          nbuf=4)
    load_idx(refa_hbm, refb_hbm, _R // _NW)
    acc_fix = _pipelined((_R // _NW) // _CH, fire, drain, ref_compute, zero,
                         nbuf=4)

    accs[pl.ds(0, _LANES)] = acc_caus
    accs[pl.ds(_LANES, _LANES)] = acc_fix
    pltpu.sync_copy(accs, out_hbm.at[wid])


_CH2 = 128
_NBUF2 = 2


@functools.partial(
    pl.kernel,
    mesh=plsc.VectorSubcoreMesh(**_SC_MESH),
    compiler_params=_SC_PARAMS,
    out_type=jax.ShapeDtypeStruct((_NW, 2 * _LANES), jnp.float32),
    scratch_types=(
        [pltpu.VMEM((_P // _NW,), jnp.int32)] * 2
        + [pltpu.VMEM((_CH2, _D), jnp.float32)] * (4 * _NBUF2)
        + [pltpu.VMEM((2 * _LANES,), jnp.float32)]
        + [pltpu.SemaphoreType.DMA] * _NBUF2
    ),
)
def _sc_pairdiff_losses(states_hbm, nstates_hbm, saa_hbm, sab_hbm,
                        out_hbm, ia_all, ib_all, *rest):
    nb = _NBUF2
    bufs = rest[:4 * nb]
    accs = rest[4 * nb]
    sems = rest[4 * nb + 1:4 * nb + 1 + nb]
    wid = lax.axis_index("s") * _NC + lax.axis_index("c")
    lane = lax.iota(jnp.int32, _LANES)
    zero = jnp.zeros((_LANES,), jnp.float32)
    bufa = bufs[0 * nb:1 * nb]
    bufb = bufs[1 * nb:2 * nb]
    bufc = bufs[2 * nb:3 * nb]
    bufd = bufs[3 * nb:4 * nb]
    per_w = _P // _NW

    pltpu.sync_copy(saa_hbm.at[pl.ds(wid * per_w, per_w)], ia_all)
    pltpu.sync_copy(sab_hbm.at[pl.ds(wid * per_w, per_w)], ib_all)

    def fire(c, s):
        ia = ia_all.at[pl.ds(c * _CH2, _CH2)]
        ib = ib_all.at[pl.ds(c * _CH2, _CH2)]
        pltpu.async_copy(states_hbm.at[ia], bufa[s], sems[s])
        pltpu.async_copy(states_hbm.at[ib], bufb[s], sems[s])
        pltpu.async_copy(nstates_hbm.at[ia], bufc[s], sems[s])
        pltpu.async_copy(nstates_hbm.at[ib], bufd[s], sems[s])

    def drain(s):
        ia = ia_all.at[pl.ds(0, _CH2)]
        pltpu.make_async_copy(states_hbm.at[ia], bufa[s], sems[s]).wait()
        pltpu.make_async_copy(states_hbm.at[ia], bufb[s], sems[s]).wait()
        pltpu.make_async_copy(states_hbm.at[ia], bufc[s], sems[s]).wait()
        pltpu.make_async_copy(states_hbm.at[ia], bufd[s], sems[s]).wait()

    cols = _cols()

    def sa_compute(s, c, carry):
        ba, bb, bc, bd = bufa[s], bufb[s], bufc[s], bufd[s]

        def grp(g, cr):
            ap, ar = cr
            rows = g * _LANES + lane
            f1 = [None] * 2
            f2 = [None] * 2
            f3 = [None] * 2
            f4 = [None] * 2
            for j in range(_D):
                sa_ = plsc.load_gather(ba, [rows, cols[j]])
                sb_ = plsc.load_gather(bb, [rows, cols[j]])
                na_ = plsc.load_gather(bc, [rows, cols[j]])
                nb_ = plsc.load_gather(bd, [rows, cols[j]])
                ds = sa_ - sb_
                da = na_ - sa_
                db = nb_ - sb_
                dd = da - db
                q = j % 2
                if f1[q] is None:
                    f1[q], f2[q], f3[q], f4[q] = ds * ds, dd * dd, da * da, db * db
                else:
                    f1[q] = f1[q] + ds * ds
                    f2[q] = f2[q] + dd * dd
                    f3[q] = f3[q] + da * da
                    f4[q] = f4[q] + db * db
            n2s = f1[0] + f1[1]
            n2d = f2[0] + f2[1]
            n2a = f3[0] + f3[1]
            n2b = f4[0] + f4[1]
            dsn = _sqrt16(n2a) - _sqrt16(n2b)
            ap = ap + dsn * dsn
            ar = ar + jnp.exp(-n2s) * n2d
            return (ap, ar)

        return lax.fori_loop(0, _CH2 // _LANES, grp, carry)

    acc_prop, acc_rep = _pipelined(per_w // _CH2, fire, drain, sa_compute,
                                   (zero, zero), nbuf=_NBUF2)

    accs[pl.ds(0, _LANES)] = acc_prop
    accs[pl.ds(_LANES, _LANES)] = acc_rep
    pltpu.sync_copy(accs, out_hbm.at[wid])


_TBLK = 131072


def _tc_body(s_ref, ns_ref, w_ref, part_ref):
    d = ns_ref[...] - s_ref[...]
    tot = jnp.sum(d * d)
    wsum = jnp.sum(jnp.abs(w_ref[...]))
    lanes = lax.broadcasted_iota(jnp.int32, (1, 8, 128), 2)
    part_ref[...] = jnp.where(lanes == 0, tot, jnp.where(lanes == 1, wsum, 0.0))


_tc_dense = pl.pallas_call(
    _tc_body,
    grid=(_N * _D // _TBLK,),
    in_specs=[
        pl.BlockSpec((_TBLK,), lambda i: (i,)),
        pl.BlockSpec((_TBLK,), lambda i: (i,)),
        pl.BlockSpec((_D, _D), lambda i: (0, 0)),
    ],
    out_specs=pl.BlockSpec((1, 8, 128), lambda i: (i, 0, 0)),
    out_shape=jax.ShapeDtypeStruct((_N * _D // _TBLK, 8, 128), jnp.float32),
)


def kernel(states, next_states, dissimilar_pairs, same_actions_pairs,
           ref_point_pairs, similar_pairs, W):
    del similar_pairs
    sc1 = _sc_states_losses(
        states,
        dissimilar_pairs[:, 0], dissimilar_pairs[:, 1],
        ref_point_pairs[:, 0], ref_point_pairs[:, 1],
    )
    sc2 = _sc_pairdiff_losses(
        states, next_states,
        same_actions_pairs[:, 0], same_actions_pairs[:, 1],
    )
    part = _tc_dense(states.reshape(-1), next_states.reshape(-1), W)
    s1 = jnp.sum(sc1.reshape(_NW, 2, _LANES), axis=(0, 2))
    s2 = jnp.sum(sc2.reshape(_NW, 2, _LANES), axis=(0, 2))
    temp_coherence = jnp.sum(part[:, 0, 0]) / _N
    l1 = part[0, 0, 1]
    return (temp_coherence
            + s1[0] / _P
            + s2[0] / _P
            + s2[1] / _P
            + s1[1] / _R
            + _L1_COEFF * l1)


# speedup vs baseline: 1.1225x; 1.0558x over previous
import functools

import jax
import jax.numpy as jnp
from jax import lax
from jax.experimental import pallas as pl
from jax.experimental.pallas import tpu as pltpu
from jax.experimental.pallas import tpu_sc as plsc

_N = 65536
_D = 64
_P = 65536
_R = 16384
_L1_COEFF = 0.001 / float(_D * _D)

_NC = 2
_NS = 16
_NW = _NC * _NS
_CH = 128
_LANES = 16

_SC_PARAMS = pltpu.CompilerParams(
    needs_layout_passes=False, use_tc_tiling_on_sc=False)
_SC_MESH = dict(core_axis_name="c", subcore_axis_name="s")


def _sqrt16(x):
    i = lax.bitcast_convert_type(x, jnp.int32)
    i = jnp.int32(0x1FBD1DF5) + lax.shift_right_logical(i, 1)
    y = lax.bitcast_convert_type(i, jnp.float32)
    for _ in range(3):
        y = 0.5 * (y + x / y)
    return y


def _cols():
    lane = lax.iota(jnp.int32, _LANES)
    return [(lane + j) & (_D - 1) for j in range(_D)]


def _pipelined(n, fire, drain, compute, carry, nbuf):
    for b in range(nbuf):
        fire(b, b)

    def body(k, cr):
        c = k * nbuf
        for b in range(nbuf):
            drain(b)
            cr = compute(b, c + b, cr)
            fire(c + nbuf + b, b)
        return cr

    carry = lax.fori_loop(0, n // nbuf - 1, body, carry)
    for b in range(nbuf):
        drain(b)
        carry = compute(b, n - nbuf + b, carry)
    return carry


@functools.partial(
    pl.kernel,
    mesh=plsc.VectorSubcoreMesh(**_SC_MESH),
    compiler_params=_SC_PARAMS,
    out_type=jax.ShapeDtypeStruct((_NW, 2 * _LANES), jnp.float32),
    scratch_types=[
        pltpu.VMEM((_P // _NW,), jnp.int32),
        pltpu.VMEM((_P // _NW,), jnp.int32),
        pltpu.VMEM((_CH, _D), jnp.float32),
        pltpu.VMEM((_CH, _D), jnp.float32),
        pltpu.VMEM((_CH, _D), jnp.float32),
        pltpu.VMEM((_CH, _D), jnp.float32),
        pltpu.VMEM((2 * _LANES,), jnp.float32),
        pltpu.SemaphoreType.DMA,
        pltpu.SemaphoreType.DMA,
    ],
)
def _sc_states_losses(states_hbm, disa_hbm, disb_hbm, refa_hbm, refb_hbm,
                      out_hbm, ia_all, ib_all, bufa0, bufa1,
                      bufb0, bufb1, accs, sem0, sem1):
    wid = lax.axis_index("s") * _NC + lax.axis_index("c")
    lane = lax.iota(jnp.int32, _LANES)
    zero = jnp.zeros((_LANES,), jnp.float32)
    bufa = (bufa0, bufa1)
    bufb = (bufb0, bufb1)
    sems = (sem0, sem1)

    def load_idx(a_hbm, b_hbm, per_w):
        pltpu.sync_copy(a_hbm.at[pl.ds(wid * per_w, per_w)],
                        ia_all.at[pl.ds(0, per_w)])
        pltpu.sync_copy(b_hbm.at[pl.ds(wid * per_w, per_w)],
                        ib_all.at[pl.ds(0, per_w)])

    def fire(c, s):
        ia = ia_all.at[pl.ds(c * _CH, _CH)]
        ib = ib_all.at[pl.ds(c * _CH, _CH)]
        pltpu.async_copy(states_hbm.at[ia], bufa[s], sems[s])
        pltpu.async_copy(states_hbm.at[ib], bufb[s], sems[s])

    def drain(s):
        ia = ia_all.at[pl.ds(0, _CH)]
        pltpu.make_async_copy(states_hbm.at[ia], bufa[s], sems[s]).wait()
        pltpu.make_async_copy(states_hbm.at[ia], bufb[s], sems[s]).wait()

    cols = _cols()

    def dist2(rows, b1, b2):
        acc = [None] * 4
        for j in range(_D):
            dv = (plsc.load_gather(b1, [rows, cols[j]])
                  - plsc.load_gather(b2, [rows, cols[j]]))
            q = j % 4
            acc[q] = dv * dv if acc[q] is None else acc[q] + dv * dv
        return (acc[0] + acc[1]) + (acc[2] + acc[3])

    def dis_compute(s, c, acc):
        def grp(g, a):
            return a + jnp.exp(-dist2(g * _LANES + lane, bufa[s], bufb[s]))

        return lax.fori_loop(0, _CH // _LANES, grp, acc)

    def ref_compute(s, c, acc):
        def grp(g, a):
            return a + dist2(g * _LANES + lane, bufa[s], bufb[s])

        return lax.fori_loop(0, _CH // _LANES, grp, acc)

    load_idx(disa_hbm, disb_hbm, _P // _NW)
    acc_caus = _pipelined((_P // _NW) // _CH, fire, drain, dis_compute, zero,
                          nbuf=2)
    load_idx(refa_hbm, refb_hbm, _R // _NW)
    acc_fix = _pipelined((_R // _NW) // _CH, fire, drain, ref_compute, zero,
                         nbuf=2)

    accs[pl.ds(0, _LANES)] = acc_caus
    accs[pl.ds(_LANES, _LANES)] = acc_fix
    pltpu.sync_copy(accs, out_hbm.at[wid])


_CH2 = 128
_NBUF2 = 2


@functools.partial(
    pl.kernel,
    mesh=plsc.VectorSubcoreMesh(**_SC_MESH),
    compiler_params=_SC_PARAMS,
    out_type=jax.ShapeDtypeStruct((_NW, 2 * _LANES), jnp.float32),
    scratch_types=(
        [pltpu.VMEM((_P // _NW,), jnp.int32)] * 2
        + [pltpu.VMEM((_CH2, _D), jnp.float32)] * (4 * _NBUF2)
        + [pltpu.VMEM((2 * _LANES,), jnp.float32)]
        + [pltpu.SemaphoreType.DMA] * _NBUF2
    ),
)
def _sc_pairdiff_losses(states_hbm, nstates_hbm, saa_hbm, sab_hbm,
                        out_hbm, ia_all, ib_all, *rest):
    nb = _NBUF2
    bufs = rest[:4 * nb]
    accs = rest[4 * nb]
    sems = rest[4 * nb + 1:4 * nb + 1 + nb]
    wid = lax.axis_index("s") * _NC + lax.axis_index("c")
    lane = lax.iota(jnp.int32, _LANES)
    zero = jnp.zeros((_LANES,), jnp.float32)
    bufa = bufs[0 * nb:1 * nb]
    bufb = bufs[1 * nb:2 * nb]
    bufc = bufs[2 * nb:3 * nb]
    bufd = bufs[3 * nb:4 * nb]
    per_w = _P // _NW

    pltpu.sync_copy(saa_hbm.at[pl.ds(wid * per_w, per_w)], ia_all)
    pltpu.sync_copy(sab_hbm.at[pl.ds(wid * per_w, per_w)], ib_all)

    def fire(c, s):
        ia = ia_all.at[pl.ds(c * _CH2, _CH2)]
        ib = ib_all.at[pl.ds(c * _CH2, _CH2)]
        pltpu.async_copy(states_hbm.at[ia], bufa[s], sems[s])
        pltpu.async_copy(states_hbm.at[ib], bufb[s], sems[s])
        pltpu.async_copy(nstates_hbm.at[ia], bufc[s], sems[s])
        pltpu.async_copy(nstates_hbm.at[ib], bufd[s], sems[s])

    def drain(s):
        ia = ia_all.at[pl.ds(0, _CH2)]
        pltpu.make_async_copy(states_hbm.at[ia], bufa[s], sems[s]).wait()
        pltpu.make_async_copy(states_hbm.at[ia], bufb[s], sems[s]).wait()
        pltpu.make_async_copy(states_hbm.at[ia], bufc[s], sems[s]).wait()
        pltpu.make_async_copy(states_hbm.at[ia], bufd[s], sems[s]).wait()

    cols = _cols()

    def sa_compute(s, c, carry):
        ba, bb, bc, bd = bufa[s], bufb[s], bufc[s], bufd[s]

        def grp(g, cr):
            ap, ar = cr
            rows = g * _LANES + lane
            f1 = [None] * 2
            f2 = [None] * 2
            f3 = [None] * 2
            f4 = [None] * 2
            for j in range(_D):
                sa_ = plsc.load_gather(ba, [rows, cols[j]])
                sb_ = plsc.load_gather(bb, [rows, cols[j]])
                na_ = plsc.load_gather(bc, [rows, cols[j]])
                nb_ = plsc.load_gather(bd, [rows, cols[j]])
                ds = sa_ - sb_
                da = na_ - sa_
                db = nb_ - sb_
                dd = da - db
                q = j % 2
                if f1[q] is None:
                    f1[q], f2[q], f3[q], f4[q] = ds * ds, dd * dd, da * da, db * db
                else:
                    f1[q] = f1[q] + ds * ds
                    f2[q] = f2[q] + dd * dd
                    f3[q] = f3[q] + da * da
                    f4[q] = f4[q] + db * db
            n2s = f1[0] + f1[1]
            n2d = f2[0] + f2[1]
            n2a = f3[0] + f3[1]
            n2b = f4[0] + f4[1]
            dsn = _sqrt16(n2a) - _sqrt16(n2b)
            ap = ap + dsn * dsn
            ar = ar + jnp.exp(-n2s) * n2d
            return (ap, ar)

        return lax.fori_loop(0, _CH2 // _LANES, grp, carry)

    acc_prop, acc_rep = _pipelined(per_w // _CH2, fire, drain, sa_compute,
                                   (zero, zero), nbuf=_NBUF2)

    accs[pl.ds(0, _LANES)] = acc_prop
    accs[pl.ds(_LANES, _LANES)] = acc_rep
    pltpu.sync_copy(accs, out_hbm.at[wid])


_TBLK = 131072


def _tc_body(s_ref, ns_ref, w_ref, part_ref):
    d = ns_ref[...] - s_ref[...]
    tot = jnp.sum(d * d)
    wsum = jnp.sum(jnp.abs(w_ref[...]))
    lanes = lax.broadcasted_iota(jnp.int32, (1, 8, 128), 2)
    part_ref[...] = jnp.where(lanes == 0, tot, jnp.where(lanes == 1, wsum, 0.0))


_tc_dense = pl.pallas_call(
    _tc_body,
    grid=(_N * _D // _TBLK,),
    in_specs=[
        pl.BlockSpec((_TBLK,), lambda i: (i,)),
        pl.BlockSpec((_TBLK,), lambda i: (i,)),
        pl.BlockSpec((_D, _D), lambda i: (0, 0)),
    ],
    out_specs=pl.BlockSpec((1, 8, 128), lambda i: (i, 0, 0)),
    out_shape=jax.ShapeDtypeStruct((_N * _D // _TBLK, 8, 128), jnp.float32),
)


def kernel(states, next_states, dissimilar_pairs, same_actions_pairs,
           ref_point_pairs, similar_pairs, W):
    del similar_pairs
    sc1 = _sc_states_losses(
        states,
        dissimilar_pairs[:, 0], dissimilar_pairs[:, 1],
        ref_point_pairs[:, 0], ref_point_pairs[:, 1],
    )
    sc2 = _sc_pairdiff_losses(
        states, next_states,
        same_actions_pairs[:, 0], same_actions_pairs[:, 1],
    )
    part = _tc_dense(states.reshape(-1), next_states.reshape(-1), W)
    s1 = jnp.sum(sc1.reshape(_NW, 2, _LANES), axis=(0, 2))
    s2 = jnp.sum(sc2.reshape(_NW, 2, _LANES), axis=(0, 2))
    temp_coherence = jnp.sum(part[:, 0, 0]) / _N
    l1 = part[0, 0, 1]
    return (temp_coherence
            + s1[0] / _P
            + s2[0] / _P
            + s2[1] / _P
            + s1[1] / _R
            + _L1_COEFF * l1)


# speedup vs baseline: 1.2631x; 1.1252x over previous
import functools

import jax
import jax.numpy as jnp
from jax import lax
from jax.experimental import pallas as pl
from jax.experimental.pallas import tpu as pltpu
from jax.experimental.pallas import tpu_sc as plsc

_N = 65536
_D = 64
_P = 65536
_R = 16384
_L1_COEFF = 0.001 / float(_D * _D)

_NC = 2
_NS = 16
_NW = _NC * _NS
_CH = 128
_LANES = 16

_SC_PARAMS = pltpu.CompilerParams(
    needs_layout_passes=False, use_tc_tiling_on_sc=False)
_SC_MESH = dict(core_axis_name="c", subcore_axis_name="s")


def _sqrt16(x):
    i = lax.bitcast_convert_type(x, jnp.int32)
    i = jnp.int32(0x1FBD1DF5) + lax.shift_right_logical(i, 1)
    y = lax.bitcast_convert_type(i, jnp.float32)
    for _ in range(3):
        y = 0.5 * (y + x / y)
    return y


def _cols():
    lane = lax.iota(jnp.int32, _LANES)
    return [(lane + j) & (_D - 1) for j in range(_D)]


def _pipelined(n, fire, drain, compute, carry, nbuf):
    for b in range(nbuf):
        fire(b, b)

    def body(k, cr):
        c = k * nbuf
        for b in range(nbuf):
            drain(b)
            cr = compute(b, c + b, cr)
            fire(c + nbuf + b, b)
        return cr

    carry = lax.fori_loop(0, n // nbuf - 1, body, carry)
    for b in range(nbuf):
        drain(b)
        carry = compute(b, n - nbuf + b, carry)
    return carry


@functools.partial(
    pl.kernel,
    mesh=plsc.VectorSubcoreMesh(**_SC_MESH),
    compiler_params=_SC_PARAMS,
    out_type=jax.ShapeDtypeStruct((_NW, 2 * _LANES), jnp.float32),
    scratch_types=[
        pltpu.VMEM((_P // _NW,), jnp.int32),
        pltpu.VMEM((_P // _NW,), jnp.int32),
        pltpu.VMEM((_CH, _D), jnp.float32),
        pltpu.VMEM((_CH, _D), jnp.float32),
        pltpu.VMEM((_CH, _D), jnp.float32),
        pltpu.VMEM((_CH, _D), jnp.float32),
        pltpu.VMEM((_LANES * _LANES,), jnp.float32),
        pltpu.VMEM((2 * _LANES,), jnp.float32),
        pltpu.SemaphoreType.DMA,
        pltpu.SemaphoreType.DMA,
    ],
)
def _sc_states_losses(states_hbm, disa_hbm, disb_hbm, refa_hbm, refb_hbm,
                      out_hbm, ia_all, ib_all, bufa0, bufa1,
                      bufb0, bufb1, fold1, accs, sem0, sem1):
    wid = lax.axis_index("s") * _NC + lax.axis_index("c")
    lane = lax.iota(jnp.int32, _LANES)
    zero = jnp.zeros((_LANES,), jnp.float32)
    bufa = (bufa0, bufa1)
    bufb = (bufb0, bufb1)
    sems = (sem0, sem1)

    def load_idx(a_hbm, b_hbm, per_w):
        pltpu.sync_copy(a_hbm.at[pl.ds(wid * per_w, per_w)],
                        ia_all.at[pl.ds(0, per_w)])
        pltpu.sync_copy(b_hbm.at[pl.ds(wid * per_w, per_w)],
                        ib_all.at[pl.ds(0, per_w)])

    def fire(c, s):
        ia = ia_all.at[pl.ds(c * _CH, _CH)]
        ib = ib_all.at[pl.ds(c * _CH, _CH)]
        pltpu.async_copy(states_hbm.at[ia], bufa[s], sems[s])
        pltpu.async_copy(states_hbm.at[ib], bufb[s], sems[s])

    def drain(s):
        ia = ia_all.at[pl.ds(0, _CH)]
        pltpu.make_async_copy(states_hbm.at[ia], bufa[s], sems[s]).wait()
        pltpu.make_async_copy(states_hbm.at[ia], bufb[s], sems[s]).wait()

    def dist2(gbase, b1, b2):
        for p in range(_LANES):
            acc = None
            for k in range(_D // _LANES):
                va = b1[gbase + p, pl.ds(k * _LANES, _LANES)]
                vb = b2[gbase + p, pl.ds(k * _LANES, _LANES)]
                dv = va - vb
                acc = dv * dv if acc is None else acc + dv * dv
            fold1[pl.ds(p * _LANES, _LANES)] = acc
        n2 = None
        for j in range(_LANES):
            v = plsc.load_gather(fold1,
                                 [lane * _LANES + ((lane + j) & (_LANES - 1))])
            n2 = v if n2 is None else n2 + v
        return n2

    def dis_compute(s, c, acc):
        def grp(g, a):
            return a + jnp.exp(-dist2(g * _LANES, bufa[s], bufb[s]))

        return lax.fori_loop(0, _CH // _LANES, grp, acc)

    def ref_compute(s, c, acc):
        def grp(g, a):
            return a + dist2(g * _LANES, bufa[s], bufb[s])

        return lax.fori_loop(0, _CH // _LANES, grp, acc)

    load_idx(disa_hbm, disb_hbm, _P // _NW)
    acc_caus = _pipelined((_P // _NW) // _CH, fire, drain, dis_compute, zero,
                          nbuf=2)
    load_idx(refa_hbm, refb_hbm, _R // _NW)
    acc_fix = _pipelined((_R // _NW) // _CH, fire, drain, ref_compute, zero,
                         nbuf=2)

    accs[pl.ds(0, _LANES)] = acc_caus
    accs[pl.ds(_LANES, _LANES)] = acc_fix
    pltpu.sync_copy(accs, out_hbm.at[wid])


_CH2 = 128
_NBUF2 = 2


@functools.partial(
    pl.kernel,
    mesh=plsc.VectorSubcoreMesh(**_SC_MESH),
    compiler_params=_SC_PARAMS,
    out_type=jax.ShapeDtypeStruct((_NW, 2 * _LANES), jnp.float32),
    scratch_types=(
        [pltpu.VMEM((_P // _NW,), jnp.int32)] * 2
        + [pltpu.VMEM((_CH2, _D), jnp.float32)] * (4 * _NBUF2)
        + [pltpu.VMEM((2 * _LANES,), jnp.float32)]
        + [pltpu.SemaphoreType.DMA] * _NBUF2
    ),
)
def _sc_pairdiff_losses(states_hbm, nstates_hbm, saa_hbm, sab_hbm,
                        out_hbm, ia_all, ib_all, *rest):
    nb = _NBUF2
    bufs = rest[:4 * nb]
    accs = rest[4 * nb]
    sems = rest[4 * nb + 1:4 * nb + 1 + nb]
    wid = lax.axis_index("s") * _NC + lax.axis_index("c")
    lane = lax.iota(jnp.int32, _LANES)
    zero = jnp.zeros((_LANES,), jnp.float32)
    bufa = bufs[0 * nb:1 * nb]
    bufb = bufs[1 * nb:2 * nb]
    bufc = bufs[2 * nb:3 * nb]
    bufd = bufs[3 * nb:4 * nb]
    per_w = _P // _NW

    pltpu.sync_copy(saa_hbm.at[pl.ds(wid * per_w, per_w)], ia_all)
    pltpu.sync_copy(sab_hbm.at[pl.ds(wid * per_w, per_w)], ib_all)

    def fire(c, s):
        ia = ia_all.at[pl.ds(c * _CH2, _CH2)]
        ib = ib_all.at[pl.ds(c * _CH2, _CH2)]
        pltpu.async_copy(states_hbm.at[ia], bufa[s], sems[s])
        pltpu.async_copy(states_hbm.at[ib], bufb[s], sems[s])
        pltpu.async_copy(nstates_hbm.at[ia], bufc[s], sems[s])
        pltpu.async_copy(nstates_hbm.at[ib], bufd[s], sems[s])

    def drain(s):
        ia = ia_all.at[pl.ds(0, _CH2)]
        pltpu.make_async_copy(states_hbm.at[ia], bufa[s], sems[s]).wait()
        pltpu.make_async_copy(states_hbm.at[ia], bufb[s], sems[s]).wait()
        pltpu.make_async_copy(states_hbm.at[ia], bufc[s], sems[s]).wait()
        pltpu.make_async_copy(states_hbm.at[ia], bufd[s], sems[s]).wait()

    cols = _cols()

    def sa_compute(s, c, carry):
        ba, bb, bc, bd = bufa[s], bufb[s], bufc[s], bufd[s]

        def grp(g, cr):
            ap, ar = cr
            rows = g * _LANES + lane
            f1 = [None] * 2
            f2 = [None] * 2
            f3 = [None] * 2
            f4 = [None] * 2
            for j in range(_D):
                sa_ = plsc.load_gather(ba, [rows, cols[j]])
                sb_ = plsc.load_gather(bb, [rows, cols[j]])
                na_ = plsc.load_gather(bc, [rows, cols[j]])
                nb_ = plsc.load_gather(bd, [rows, cols[j]])
                ds = sa_ - sb_
                da = na_ - sa_
                db = nb_ - sb_
                dd = da - db
                q = j % 2
                if f1[q] is None:
                    f1[q], f2[q], f3[q], f4[q] = ds * ds, dd * dd, da * da, db * db
                else:
                    f1[q] = f1[q] + ds * ds
                    f2[q] = f2[q] + dd * dd
                    f3[q] = f3[q] + da * da
                    f4[q] = f4[q] + db * db
            n2s = f1[0] + f1[1]
            n2d = f2[0] + f2[1]
            n2a = f3[0] + f3[1]
            n2b = f4[0] + f4[1]
            dsn = _sqrt16(n2a) - _sqrt16(n2b)
            ap = ap + dsn * dsn
            ar = ar + jnp.exp(-n2s) * n2d
            return (ap, ar)

        return lax.fori_loop(0, _CH2 // _LANES, grp, carry)

    acc_prop, acc_rep = _pipelined(per_w // _CH2, fire, drain, sa_compute,
                                   (zero, zero), nbuf=_NBUF2)

    accs[pl.ds(0, _LANES)] = acc_prop
    accs[pl.ds(_LANES, _LANES)] = acc_rep
    pltpu.sync_copy(accs, out_hbm.at[wid])


_TBLK = 131072


def _tc_body(s_ref, ns_ref, w_ref, part_ref):
    d = ns_ref[...] - s_ref[...]
    tot = jnp.sum(d * d)
    wsum = jnp.sum(jnp.abs(w_ref[...]))
    lanes = lax.broadcasted_iota(jnp.int32, (1, 8, 128), 2)
    part_ref[...] = jnp.where(lanes == 0, tot, jnp.where(lanes == 1, wsum, 0.0))


_tc_dense = pl.pallas_call(
    _tc_body,
    grid=(_N * _D // _TBLK,),
    in_specs=[
        pl.BlockSpec((_TBLK,), lambda i: (i,)),
        pl.BlockSpec((_TBLK,), lambda i: (i,)),
        pl.BlockSpec((_D, _D), lambda i: (0, 0)),
    ],
    out_specs=pl.BlockSpec((1, 8, 128), lambda i: (i, 0, 0)),
    out_shape=jax.ShapeDtypeStruct((_N * _D // _TBLK, 8, 128), jnp.float32),
)


def kernel(states, next_states, dissimilar_pairs, same_actions_pairs,
           ref_point_pairs, similar_pairs, W):
    del similar_pairs
    sc1 = _sc_states_losses(
        states,
        dissimilar_pairs[:, 0], dissimilar_pairs[:, 1],
        ref_point_pairs[:, 0], ref_point_pairs[:, 1],
    )
    sc2 = _sc_pairdiff_losses(
        states, next_states,
        same_actions_pairs[:, 0], same_actions_pairs[:, 1],
    )
    part = _tc_dense(states.reshape(-1), next_states.reshape(-1), W)
    s1 = jnp.sum(sc1.reshape(_NW, 2, _LANES), axis=(0, 2))
    s2 = jnp.sum(sc2.reshape(_NW, 2, _LANES), axis=(0, 2))
    temp_coherence = jnp.sum(part[:, 0, 0]) / _N
    l1 = part[0, 0, 1]
    return (temp_coherence
            + s1[0] / _P
            + s2[0] / _P
            + s2[1] / _P
            + s1[1] / _R
            + _L1_COEFF * l1)
